# Initial kernel scaffold; baseline (speedup 1.0000x reference)
import jax
import jax.numpy as jnp
from jax.experimental import pallas as pl


def kernel(memory, node_idxs, values):
    raise NotImplementedError("write your pallas kernel here")


# speedup vs baseline: 3.7345x; 3.7345x over previous
import functools

import jax
import jax.numpy as jnp
from jax import lax
from jax.experimental import pallas as pl
from jax.experimental.pallas import tpu as pltpu
from jax.experimental.pallas import tpu_sc as plsc

N_NODES = 100000
ROW = 512
BATCH = 16384
L = 16
NW = 32
RPW = N_NODES // NW
RPW_PAD = RPW + (-RPW % L)
LIST_LEN = RPW_PAD + L

@functools.cache
def _build_sc_scatter():
    mesh = plsc.VectorSubcoreMesh(
        core_axis_name="c", subcore_axis_name="s", num_cores=2, num_subcores=16
    )
    return pl.kernel(
        _sc_scatter_body,
        out_type=(),
        mesh=mesh,
        compiler_params=pltpu.CompilerParams(needs_layout_passes=False),
        scratch_types=[
            pltpu.VMEM((BATCH,), jnp.int32),
            pltpu.VMEM((RPW_PAD,), jnp.int32),
            pltpu.VMEM((LIST_LEN,), jnp.int32),
            pltpu.VMEM((LIST_LEN,), jnp.int32),
            pltpu.VMEM((L, ROW), jnp.float32),
            pltpu.SemaphoreType.DMA,
            pltpu.SemaphoreType.DMA,
        ],
    )


def _sc_scatter_body(mem, idx_hbm, vals_hbm, idx_v, aux, wlist, rlist, gbuf,
                     gsem, ssem):
    wid = lax.axis_index("c") * 16 + lax.axis_index("s")
    lo = (wid * RPW).astype(jnp.int32)
    lane = lax.iota(jnp.int32, L)

    pltpu.sync_copy(idx_hbm, idx_v)

    neg1 = jnp.full((L,), -1, jnp.int32)

    def init_body(i, carry):
        aux[pl.ds(i * L, L)] = neg1
        return carry

    lax.fori_loop(0, RPW_PAD // L, init_body, 0)

    def fill_body(i, carry):
        v = idx_v[pl.ds(i * L, L)]
        owned = (v >= lo) & (v < lo + RPW)
        _, last = plsc.scan_count(v, mask=owned)
        win = last & owned
        local = jnp.where(win, v - lo, 0)
        pos = (i * L + lane).astype(jnp.int32)
        plsc.store_scatter(aux, [local], pos, mask=win)
        return carry

    lax.fori_loop(0, BATCH // L, fill_body, 0)

    def comp_body(c, off):
        a = aux[pl.ds(c * L, L)]
        m = a >= 0
        rows = lo + c * L + lane
        plsc.store_compressed(wlist.at[pl.ds(off, L)], a, mask=m)
        plsc.store_compressed(rlist.at[pl.ds(off, L)], rows, mask=m)
        return off + jnp.sum(m.astype(jnp.int32))

    n = lax.fori_loop(0, RPW_PAD // L, comp_body, jnp.int32(0))

    @pl.when(n >= L)
    def _():
        nch = (n + L - 1) >> 4

        def dma_body(k, carry):
            o = jnp.minimum(k * L, n - L)
            wv = wlist[pl.ds(o, L)]
            rv = rlist[pl.ds(o, L)]
            pltpu.async_copy(vals_hbm.at[wv], gbuf, gsem).wait()
            pltpu.async_copy(gbuf, mem.at[rv], ssem).wait()
            return carry

        lax.fori_loop(0, nch, dma_body, 0)

    @pl.when((n > 0) & (n < L))
    def _():
        wv = wlist[pl.ds(0, L)]
        rv = rlist[pl.ds(0, L)]

        def tail_body(i, carry):
            @pl.when(i < n)
            def _():
                wsc = jnp.max(jnp.where(lane == i, wv, -1))
                rsc = jnp.max(jnp.where(lane == i, rv, -1))
                pltpu.sync_copy(vals_hbm.at[pl.ds(wsc, 1)],
                                gbuf.at[pl.ds(0, 1)])
                pltpu.sync_copy(gbuf.at[pl.ds(0, 1)],
                                mem.at[pl.ds(rsc, 1)])
            return carry

        lax.fori_loop(0, L, tail_body, 0)


def kernel(memory, node_idxs, values):
    mem2 = memory.reshape(N_NODES, ROW)
    vals2 = values.reshape(BATCH, ROW)
    idx = node_idxs.astype(jnp.int32)
    ref = jax.new_ref(mem2)
    _build_sc_scatter()(ref, idx, vals2)
    return ref[...].reshape(memory.shape)
